# baseline (speedup 1.0000x reference)
import functools

import jax
import jax.numpy as jnp
from jax import lax
from jax.experimental import pallas as pl
from jax.experimental.pallas import tpu as pltpu
from jax.experimental.pallas import tpu_sc as plsc

BATCH = 16384
D = 64
NC, NS = 2, 16
NW = NC * NS
B_PER_W = BATCH // NW
CHUNK = 128
NCHUNK = B_PER_W // CHUNK


def _gather_both(user_ids3, movie_ids3, user_table, movie_table):
    mesh = plsc.VectorSubcoreMesh(core_axis_name="c", subcore_axis_name="s")

    @functools.partial(
        pl.kernel,
        mesh=mesh,
        compiler_params=pltpu.CompilerParams(use_tc_tiling_on_sc=False),
        out_type=[
            jax.ShapeDtypeStruct((BATCH, D), jnp.float32),
            jax.ShapeDtypeStruct((BATCH, D), jnp.float32),
        ],
        scratch_types=[
            pltpu.VMEM((NCHUNK, CHUNK), jnp.int32),
            pltpu.VMEM((NCHUNK, CHUNK), jnp.int32),
            pltpu.VMEM((B_PER_W, D), jnp.float32),
            pltpu.VMEM((B_PER_W, D), jnp.float32),
            pltpu.SemaphoreType.DMA,
            pltpu.SemaphoreType.DMA,
        ],
    )
    def gather_k(uid_hbm, mid_hbm, ut_hbm, mt_hbm, uemb_hbm, memb_hbm,
                 uidx_v, midx_v, urows_v, mrows_v, usem, msem):
        wid = lax.axis_index("s") * NC + lax.axis_index("c")
        base = wid * B_PER_W
        pltpu.sync_copy(uid_hbm.at[wid], uidx_v)
        pltpu.sync_copy(mid_hbm.at[wid], midx_v)
        copies = []
        for j in range(NCHUNK):
            copies.append(pltpu.async_copy(
                ut_hbm.at[uidx_v.at[j]],
                urows_v.at[pl.ds(j * CHUNK, CHUNK)], usem))
            copies.append(pltpu.async_copy(
                mt_hbm.at[midx_v.at[j]],
                mrows_v.at[pl.ds(j * CHUNK, CHUNK)], msem))
        for c in copies:
            c.wait()
        pltpu.sync_copy(urows_v, uemb_hbm.at[pl.ds(base, B_PER_W)])
        pltpu.sync_copy(mrows_v, memb_hbm.at[pl.ds(base, B_PER_W)])

    return gather_k(user_ids3, movie_ids3, user_table, movie_table)


def _mlp_block(u_ref, m_ref, w1u_ref, w1m_ref, b1_ref, w2_ref, b2_ref,
               w3_ref, b3_ref, o_ref):
    h = (jnp.dot(u_ref[...], w1u_ref[...], preferred_element_type=jnp.float32)
         + jnp.dot(m_ref[...], w1m_ref[...], preferred_element_type=jnp.float32)
         + b1_ref[...])
    h = jnp.maximum(h, 0.0)
    h = jnp.dot(h, w2_ref[...], preferred_element_type=jnp.float32) + b2_ref[...]
    h = jnp.maximum(h, 0.0)
    r = jnp.dot(h, w3_ref[...], preferred_element_type=jnp.float32) + b3_ref[...]
    o_ref[...] = jax.nn.sigmoid(r)


def _mlp(user_emb, movie_emb, W1, b1, W2, b2, W3, b3):
    NB = 2048
    grid = (BATCH // NB,)

    def full(shape):
        return pl.BlockSpec(shape, lambda i: (0,) * len(shape))

    return pl.pallas_call(
        _mlp_block,
        grid=grid,
        in_specs=[
            pl.BlockSpec((NB, D), lambda i: (i, 0)),
            pl.BlockSpec((NB, D), lambda i: (i, 0)),
            full((D, 128)),
            full((D, 128)),
            full((1, 128)),
            full((128, D)),
            full((1, D)),
            full((D, 1)),
            full((1, 1)),
        ],
        out_specs=pl.BlockSpec((NB, 1), lambda i: (i, 0)),
        out_shape=jax.ShapeDtypeStruct((BATCH, 1), jnp.float32),
    )(user_emb, movie_emb, W1[:D], W1[D:], b1.reshape(1, 128),
      W2, b2.reshape(1, D), W3, b3.reshape(1, 1))


def kernel(user_ids, movie_ids, user_table, movie_table,
           W1, b1, W2, b2, W3, b3):
    uid3 = user_ids.astype(jnp.int32).reshape(NW, NCHUNK, CHUNK)
    mid3 = movie_ids.astype(jnp.int32).reshape(NW, NCHUNK, CHUNK)
    user_emb, movie_emb = _gather_both(uid3, mid3, user_table, movie_table)
    rating = _mlp(user_emb, movie_emb, W1, b1, W2, b2, W3, b3)
    return rating.reshape(BATCH)


# speedup vs baseline: 1.6641x; 1.6641x over previous
import functools

import jax
import jax.numpy as jnp
from jax import lax
from jax.experimental import pallas as pl
from jax.experimental.pallas import tpu as pltpu
from jax.experimental.pallas import tpu_sc as plsc

BATCH = 16384
D = 64
NC, NS = 2, 16
NW = NC * NS
B_PER_W = BATCH // NW


def _gather_concat(user_ids, movie_ids, user_table, movie_table):
    mesh = plsc.VectorSubcoreMesh(core_axis_name="c", subcore_axis_name="s")

    @functools.partial(
        pl.kernel,
        mesh=mesh,
        compiler_params=pltpu.CompilerParams(needs_layout_passes=False),
        out_type=jax.ShapeDtypeStruct((BATCH, 2 * D), jnp.float32),
        scratch_types=[
            pltpu.VMEM_SHARED((NS, B_PER_W), jnp.int32),
            pltpu.SMEM((2 * B_PER_W,), jnp.int32),
            pltpu.VMEM((B_PER_W, 2 * D), jnp.float32),
            pltpu.SemaphoreType.DMA,
            pltpu.SemaphoreType.DMA,
        ],
    )
    def gather_k(uid_hbm, mid_hbm, ut_hbm, mt_hbm, comb_hbm, sh, sm, comb,
                 usem, msem):
        s = lax.axis_index("s")
        wid = s * NC + lax.axis_index("c")
        base = wid * B_PER_W
        pltpu.sync_copy(uid_hbm.at[pl.ds(base, B_PER_W)], sh.at[s])
        pltpu.sync_copy(sh.at[s], sm.at[pl.ds(0, B_PER_W)])
        pltpu.sync_copy(mid_hbm.at[pl.ds(base, B_PER_W)], sh.at[s])
        pltpu.sync_copy(sh.at[s], sm.at[pl.ds(B_PER_W, B_PER_W)])

        def issue(kk, _):
            pltpu.async_copy(ut_hbm.at[sm[kk]],
                             comb.at[kk, pl.ds(0, D)], usem)
            pltpu.async_copy(mt_hbm.at[sm[B_PER_W + kk]],
                             comb.at[kk, pl.ds(D, D)], msem)
            return 0
        lax.fori_loop(0, B_PER_W, issue, 0, unroll=4)

        def drain(kk, _):
            pltpu.make_async_copy(ut_hbm.at[0],
                                  comb.at[kk, pl.ds(0, D)], usem).wait()
            pltpu.make_async_copy(mt_hbm.at[0],
                                  comb.at[kk, pl.ds(D, D)], msem).wait()
            return 0
        lax.fori_loop(0, B_PER_W, drain, 0, unroll=4)

        pltpu.sync_copy(comb, comb_hbm.at[pl.ds(base, B_PER_W)])

    return gather_k(user_ids, movie_ids, user_table, movie_table)


def _mlp_block(x_ref, w1_ref, b1_ref, w2_ref, b2_ref, w3_ref, b3_ref, o_ref):
    h = (jnp.dot(x_ref[...], w1_ref[...], preferred_element_type=jnp.float32)
         + b1_ref[...])
    h = jnp.maximum(h, 0.0)
    h = jnp.dot(h, w2_ref[...], preferred_element_type=jnp.float32) + b2_ref[...]
    h = jnp.maximum(h, 0.0)
    r = jnp.dot(h, w3_ref[...], preferred_element_type=jnp.float32) + b3_ref[...]
    o_ref[...] = jax.nn.sigmoid(r)


def _mlp(comb, W1, b1, W2, b2, W3, b3):
    NB = 2048
    grid = (BATCH // NB,)

    def full(shape):
        return pl.BlockSpec(shape, lambda i: (0,) * len(shape))

    return pl.pallas_call(
        _mlp_block,
        grid=grid,
        in_specs=[
            pl.BlockSpec((NB, 2 * D), lambda i: (i, 0)),
            full((2 * D, 128)),
            full((1, 128)),
            full((128, D)),
            full((1, D)),
            full((D, 1)),
            full((1, 1)),
        ],
        out_specs=pl.BlockSpec((NB, 1), lambda i: (i, 0)),
        out_shape=jax.ShapeDtypeStruct((BATCH, 1), jnp.float32),
    )(comb, W1, b1.reshape(1, 128), W2, b2.reshape(1, D), W3,
      b3.reshape(1, 1))


def kernel(user_ids, movie_ids, user_table, movie_table,
           W1, b1, W2, b2, W3, b3):
    comb = _gather_concat(user_ids.astype(jnp.int32),
                          movie_ids.astype(jnp.int32),
                          user_table, movie_table)
    rating = _mlp(comb, W1, b1, W2, b2, W3, b3)
    return rating.reshape(BATCH)
